# baseline (speedup 1.0000x reference)
import numpy as np

import jax
import jax.numpy as jnp
from jax.experimental import pallas as pl
from jax.experimental.pallas import tpu as pltpu

_G = 4


def _pool2x2(act_ref, outer, inner, c):
    m = (outer * inner) // 2
    pw = jnp.maximum(act_ref[pl.ds(0, m, 2), :], act_ref[pl.ds(1, m, 2), :])
    return jnp.max(pw.reshape(outer // 2, 2, inner // 2, c), axis=1).reshape(
        (outer // 2) * (inner // 2), c)


def _row_patches(x, rows, period, c):
    col = jax.lax.broadcasted_iota(jnp.int32, (rows, c), 0) % period
    zeros_row = jnp.zeros((1, c), x.dtype)
    prev = jnp.concatenate([zeros_row, x[: rows - 1, :]], axis=0)
    prev = jnp.where(col == 0, jnp.bfloat16(0), prev)
    nxt = jnp.concatenate([x[1:, :], zeros_row], axis=0)
    nxt = jnp.where(col == period - 1, jnp.bfloat16(0), nxt)
    return jnp.concatenate([prev, x, nxt], axis=1)


def _cascade_steps(n):
    steps = []
    while n > 1:
        r = 8 if n % 8 == 0 else (4 if n % 4 == 0 else 2)
        steps.append(r)
        n //= r
    return steps


def _cascade_to_pixel_major(a, rows, lanes, steps, scratches):
    for radix, s in zip(steps, scratches):
        m = lanes // radix
        for j in range(radix):
            s[pl.ds(j * rows, rows), :] = a[:, j * m:(j + 1) * m]
        a = s[...]
        rows, lanes = rows * radix, m
    return a


def _cascade_order(n):
    order = np.arange(n)[None, :]
    for radix in _cascade_steps(n):
        m = order.shape[1] // radix
        order = np.vstack([order[:, j * m:(j + 1) * m] for j in range(radix)])
    return order[:, 0]


def kernel(img_nchw, meta, w1, b1, w2, b2, w3, b3, w_img_t,
           w_meta, b_meta, w_meta_out, b_out):
    B, Cin, H, W = img_nchw.shape
    C1 = w1.shape[1]
    C2 = w2.shape[1]
    C3 = w3.shape[1]
    H2, W2 = H // 2, W // 2
    H3, W3 = H // 4, W // 4
    H4, W4 = H // 8, W // 8
    NC = w_img_t.shape[0]
    NM = meta.shape[1]
    G = _G
    I2, I3, I4 = G * H2, G * H3, G * H4
    M2, M3 = W2 * I2, W3 * I3
    R = H4 * W4

    w1r = w1.reshape(3, 3, Cin, C1)
    shift = jnp.stack([jnp.eye(W, k=1 - dw, dtype=jnp.float32)
                       for dw in range(3)])
    t1 = jnp.einsum('xyco,yiw->xciwo', w1r, shift)
    t1 = t1.reshape(3, Cin, W, W2, 2, C1).transpose(0, 1, 2, 4, 3, 5)
    inv = np.argsort(_cascade_order(W2))
    t1 = t1[:, :, :, :, inv, :]
    t1 = t1.reshape(3 * Cin * W, 2 * W2 * C1).astype(jnp.bfloat16)
    b1_full = jnp.tile(b1, (1, W2))

    w2r = (w2.reshape(3, 3, C1, C2).transpose(1, 0, 2, 3)
           .reshape(3, 3 * C1, C2).astype(jnp.bfloat16))
    w3r = (w3.reshape(3, 3, C2, C3).transpose(1, 0, 2, 3)
           .reshape(3, 3 * C2, C3).astype(jnp.bfloat16))
    wi_t = (w_img_t.reshape(NC, H4, W4, C3).transpose(0, 2, 1, 3)
            .reshape(NC, W4, 1, H4, C3))
    wi_exp = jnp.broadcast_to(wi_t, (NC, W4, G, H4, C3)).reshape(
        NC, W4 * G * H4, C3)
    meta3 = meta.reshape(B, 1, NM)

    def body(img_ref, meta_ref, t1_ref, b1_ref, w2_ref, b2_ref, w3_ref,
             b3_ref, wi_ref, wm_ref, bm_ref, wmo_ref, bo_ref, o_ref,
             p2_s, act2_s, p3_s, act3_s, *casc_s):
        zrow = jnp.zeros((1, W), jnp.bfloat16)
        ev_blocks, od_blocks = [], []
        for g in range(G):
            evens = [img_ref[g, c, pl.ds(0, H2, 2), :].astype(jnp.bfloat16)
                     for c in range(Cin)]
            odds = [img_ref[g, c, pl.ds(1, H2, 2), :].astype(jnp.bfloat16)
                    for c in range(Cin)]
            ecols, ocols = [], []
            for dh in range(3):
                for c in range(Cin):
                    if dh == 0:
                        ecols.append(jnp.concatenate(
                            [zrow, odds[c][: H2 - 1, :]], axis=0))
                        ocols.append(evens[c])
                    elif dh == 1:
                        ecols.append(evens[c])
                        ocols.append(odds[c])
                    else:
                        ecols.append(odds[c])
                        ocols.append(jnp.concatenate(
                            [evens[c][1:, :], zrow], axis=0))
            ev_blocks.append(jnp.concatenate(ecols, axis=1))
            od_blocks.append(jnp.concatenate(ocols, axis=1))
        xs = jnp.concatenate(ev_blocks + od_blocks, axis=0)
        a1 = jnp.dot(xs, t1_ref[...], preferred_element_type=jnp.float32)
        pw = jnp.maximum(a1[:, : W2 * C1], a1[:, W2 * C1:])
        ph = jnp.maximum(pw[: G * H2, :], pw[G * H2:, :])
        ph = jnp.maximum(ph + b1_ref[...], 0.0)
        steps = _cascade_steps(W2)
        a_pre = _cascade_to_pixel_major(ph.astype(jnp.bfloat16), G * H2,
                                        W2 * C1, steps[:-1], casc_s)
        rows_pre = G * H2
        for r_ in steps[:-1]:
            rows_pre *= r_

        p2_s[pl.ds(0, I2 + 1), :] = jnp.zeros((I2 + 1, 3 * C1), jnp.bfloat16)
        p2_s[pl.ds(I2 + M2 - 1, I2 + 1), :] = jnp.zeros(
            (I2 + 1, 3 * C1), jnp.bfloat16)
        hpos = jax.lax.broadcasted_iota(jnp.int32, (rows_pre, C1), 0) % H2
        for j in range(steps[-1]):
            slab = a_pre[:, j * C1:(j + 1) * C1]
            base = I2 + j * rows_pre
            p2_s[pl.ds(base, rows_pre), pl.ds(C1, C1)] = slab
            p2_s[pl.ds(base + 1, rows_pre), pl.ds(0, C1)] = jnp.where(
                hpos == H2 - 1, jnp.bfloat16(0), slab)
            p2_s[pl.ds(base - 1, rows_pre), pl.ds(2 * C1, C1)] = jnp.where(
                hpos == 0, jnp.bfloat16(0), slab)
        a2 = (jnp.dot(p2_s[pl.ds(0, M2), :], w2_ref[0],
                      preferred_element_type=jnp.float32)
              + jnp.dot(p2_s[pl.ds(I2, M2), :], w2_ref[1],
                        preferred_element_type=jnp.float32)
              + jnp.dot(p2_s[pl.ds(2 * I2, M2), :], w2_ref[2],
                        preferred_element_type=jnp.float32))
        act2_s[...] = a2
        x3 = jnp.maximum(_pool2x2(act2_s, W2, I2, C2) + b2_ref[...],
                         0.0).astype(jnp.bfloat16)

        p3_s[pl.ds(0, I3), :] = jnp.zeros((I3, 3 * C2), jnp.bfloat16)
        p3_s[pl.ds(I3 + M3, I3), :] = jnp.zeros((I3, 3 * C2), jnp.bfloat16)
        p3_s[pl.ds(I3, M3), :] = _row_patches(x3, M3, H3, C2)
        a3 = (jnp.dot(p3_s[pl.ds(0, M3), :], w3_ref[0],
                      preferred_element_type=jnp.float32)
              + jnp.dot(p3_s[pl.ds(I3, M3), :], w3_ref[1],
                        preferred_element_type=jnp.float32)
              + jnp.dot(p3_s[pl.ds(2 * I3, M3), :], w3_ref[2],
                        preferred_element_type=jnp.float32))
        act3_s[...] = a3
        xf = jnp.maximum(_pool2x2(act3_s, W3, I3, C3) + b3_ref[...],
                         0.0)

        ils = []
        for c in range(NC):
            prod = wi_ref[c] * xf
            t = jnp.sum(prod.reshape(W4, G * H4, C3), axis=0)
            u = jnp.sum(t.reshape(G, H4, C3), axis=1)
            ils.append(jnp.sum(u, axis=1, keepdims=True))
        il = jnp.concatenate(ils, axis=1)
        mo = jnp.maximum(
            jnp.dot(meta_ref[:, 0, :], wm_ref[...],
                    preferred_element_type=jnp.float32) + bm_ref[...], 0.0)
        ml = jnp.dot(mo, wmo_ref[...], preferred_element_type=jnp.float32)
        o_ref[:, 0, :] = ml + bo_ref[...] + il

    casc_shapes = []
    r_, l_ = G * H2, W2 * C1
    for radix in _cascade_steps(W2)[:-1]:
        r_, l_ = r_ * radix, l_ // radix
        casc_shapes.append(pltpu.VMEM((r_, l_), jnp.bfloat16))

    const2 = lambda b: (0, 0)
    const3 = lambda b: (0, 0, 0)
    out = pl.pallas_call(
        body,
        out_shape=jax.ShapeDtypeStruct((B, 1, NC), jnp.float32),
        grid=(B // G,),
        in_specs=[
            pl.BlockSpec((G, Cin, H, W), lambda b: (b, 0, 0, 0)),
            pl.BlockSpec((G, 1, NM), lambda b: (b, 0, 0)),
            pl.BlockSpec(t1.shape, const2),
            pl.BlockSpec(b1_full.shape, const2),
            pl.BlockSpec(w2r.shape, const3),
            pl.BlockSpec(b2.shape, const2),
            pl.BlockSpec(w3r.shape, const3),
            pl.BlockSpec(b3.shape, const2),
            pl.BlockSpec(wi_exp.shape, const3),
            pl.BlockSpec(w_meta.shape, const2),
            pl.BlockSpec(b_meta.shape, const2),
            pl.BlockSpec(w_meta_out.shape, const2),
            pl.BlockSpec(b_out.shape, const2),
        ],
        out_specs=pl.BlockSpec((G, 1, NC), lambda b: (b, 0, 0)),
        scratch_shapes=[
            pltpu.VMEM((M2 + 2 * I2, 3 * C1), jnp.bfloat16),
            pltpu.VMEM((M2, C2), jnp.float32),
            pltpu.VMEM((M3 + 2 * I3, 3 * C2), jnp.bfloat16),
            pltpu.VMEM((M3, C3), jnp.float32),
        ] + casc_shapes,
        compiler_params=pltpu.CompilerParams(
            dimension_semantics=("parallel",),
            vmem_limit_bytes=56 * 1024 * 1024),
    )(img_nchw, meta3, t1, b1_full, w2r, b2, w3r, b3, wi_exp,
      w_meta, b_meta, w_meta_out, b_out)
    return out.reshape(B, NC)


# speedup vs baseline: 1.0496x; 1.0496x over previous
import numpy as np

import jax
import jax.numpy as jnp
from jax.experimental import pallas as pl
from jax.experimental.pallas import tpu as pltpu

_G = 4


def _pool2x2(act_ref, outer, inner, c):
    m = (outer * inner) // 2
    pw = jnp.maximum(act_ref[pl.ds(0, m, 2), :], act_ref[pl.ds(1, m, 2), :])
    return jnp.max(pw.reshape(outer // 2, 2, inner // 2, c), axis=1).reshape(
        (outer // 2) * (inner // 2), c)


def _row_patches(x, rows, period, c):
    col = jax.lax.broadcasted_iota(jnp.int32, (rows, c), 0) % period
    zeros_row = jnp.zeros((1, c), x.dtype)
    prev = jnp.concatenate([zeros_row, x[: rows - 1, :]], axis=0)
    prev = jnp.where(col == 0, jnp.bfloat16(0), prev)
    nxt = jnp.concatenate([x[1:, :], zeros_row], axis=0)
    nxt = jnp.where(col == period - 1, jnp.bfloat16(0), nxt)
    return jnp.concatenate([prev, x, nxt], axis=1)


def _cascade_steps(n):
    steps = []
    while n > 1:
        r = 4 if n % 4 == 0 else 2
        steps.append(r)
        n //= r
    return steps


def _cascade_to_pixel_major(a, rows, lanes, steps, scratches):
    for radix, s in zip(steps, scratches):
        m = lanes // radix
        for j in range(radix):
            s[pl.ds(j * rows, rows), :] = a[:, j * m:(j + 1) * m]
        a = s[...]
        rows, lanes = rows * radix, m
    return a


def _cascade_order(n):
    order = np.arange(n)[None, :]
    for radix in _cascade_steps(n):
        m = order.shape[1] // radix
        order = np.vstack([order[:, j * m:(j + 1) * m] for j in range(radix)])
    return order[:, 0]


def kernel(img_nchw, meta, w1, b1, w2, b2, w3, b3, w_img_t,
           w_meta, b_meta, w_meta_out, b_out):
    B, Cin, H, W = img_nchw.shape
    C1 = w1.shape[1]
    C2 = w2.shape[1]
    C3 = w3.shape[1]
    H2, W2 = H // 2, W // 2
    H3, W3 = H // 4, W // 4
    H4, W4 = H // 8, W // 8
    NC = w_img_t.shape[0]
    NM = meta.shape[1]
    G = _G
    I2, I3, I4 = G * H2, G * H3, G * H4
    M2, M3 = W2 * I2, W3 * I3
    R = H4 * W4

    w1r = w1.reshape(3, 3, Cin, C1)
    shift = jnp.stack([jnp.eye(W, k=1 - dw, dtype=jnp.float32)
                       for dw in range(3)])
    t1 = jnp.einsum('xyco,yiw->xciwo', w1r, shift)
    t1 = t1.reshape(3, Cin, W, W2, 2, C1).transpose(0, 1, 2, 4, 3, 5)
    inv = np.argsort(_cascade_order(W2))
    t1 = t1[:, :, :, :, inv, :]
    t1 = t1.reshape(3 * Cin * W, 2 * W2 * C1).astype(jnp.bfloat16)
    b1_full = jnp.tile(b1, (1, W2))

    w2r = (w2.reshape(3, 3, C1, C2).transpose(1, 0, 2, 3)
           .reshape(3, 3 * C1, C2).astype(jnp.bfloat16))
    w3r = (w3.reshape(3, 3, C2, C3).transpose(1, 0, 2, 3)
           .reshape(3, 3 * C2, C3).astype(jnp.bfloat16))
    wi_t = (w_img_t.reshape(NC, H4, W4, C3).transpose(0, 2, 1, 3)
            .reshape(NC, W4, 1, H4, C3))
    wi_exp = jnp.broadcast_to(wi_t, (NC, W4, G, H4, C3)).reshape(
        NC, W4 * G * H4, C3)
    meta3 = meta.reshape(B, 1, NM)

    def body(img_ref, meta_ref, t1_ref, b1_ref, w2_ref, b2_ref, w3_ref,
             b3_ref, wi_ref, wm_ref, bm_ref, wmo_ref, bo_ref, o_ref,
             p2_s, act2_s, p3_s, act3_s, *casc_s):
        zrow = jnp.zeros((1, W), jnp.bfloat16)
        ev_blocks, od_blocks = [], []
        for g in range(G):
            evens = [img_ref[g, c, pl.ds(0, H2, 2), :].astype(jnp.bfloat16)
                     for c in range(Cin)]
            odds = [img_ref[g, c, pl.ds(1, H2, 2), :].astype(jnp.bfloat16)
                    for c in range(Cin)]
            ecols, ocols = [], []
            for dh in range(3):
                for c in range(Cin):
                    if dh == 0:
                        ecols.append(jnp.concatenate(
                            [zrow, odds[c][: H2 - 1, :]], axis=0))
                        ocols.append(evens[c])
                    elif dh == 1:
                        ecols.append(evens[c])
                        ocols.append(odds[c])
                    else:
                        ecols.append(odds[c])
                        ocols.append(jnp.concatenate(
                            [evens[c][1:, :], zrow], axis=0))
            ev_blocks.append(jnp.concatenate(ecols, axis=1))
            od_blocks.append(jnp.concatenate(ocols, axis=1))
        xs = jnp.concatenate(ev_blocks + od_blocks, axis=0)
        a1 = jnp.dot(xs, t1_ref[...], preferred_element_type=jnp.float32)
        pw = jnp.maximum(a1[:, : W2 * C1], a1[:, W2 * C1:])
        ph = jnp.maximum(pw[: G * H2, :], pw[G * H2:, :])
        ph = jnp.maximum(ph + b1_ref[...], 0.0)
        x2 = _cascade_to_pixel_major(ph.astype(jnp.bfloat16), G * H2,
                                     W2 * C1, _cascade_steps(W2),
                                     casc_s)

        p2_s[pl.ds(0, I2), :] = jnp.zeros((I2, 3 * C1), jnp.bfloat16)
        p2_s[pl.ds(I2 + M2, I2), :] = jnp.zeros((I2, 3 * C1), jnp.bfloat16)
        p2_s[pl.ds(I2, M2), :] = _row_patches(x2, M2, H2, C1)
        a2 = (jnp.dot(p2_s[pl.ds(0, M2), :], w2_ref[0],
                      preferred_element_type=jnp.float32)
              + jnp.dot(p2_s[pl.ds(I2, M2), :], w2_ref[1],
                        preferred_element_type=jnp.float32)
              + jnp.dot(p2_s[pl.ds(2 * I2, M2), :], w2_ref[2],
                        preferred_element_type=jnp.float32))
        act2_s[...] = jnp.maximum(a2 + b2_ref[...], 0.0)
        x3 = _pool2x2(act2_s, W2, I2, C2).astype(jnp.bfloat16)

        p3_s[pl.ds(0, I3), :] = jnp.zeros((I3, 3 * C2), jnp.bfloat16)
        p3_s[pl.ds(I3 + M3, I3), :] = jnp.zeros((I3, 3 * C2), jnp.bfloat16)
        p3_s[pl.ds(I3, M3), :] = _row_patches(x3, M3, H3, C2)
        a3 = (jnp.dot(p3_s[pl.ds(0, M3), :], w3_ref[0],
                      preferred_element_type=jnp.float32)
              + jnp.dot(p3_s[pl.ds(I3, M3), :], w3_ref[1],
                        preferred_element_type=jnp.float32)
              + jnp.dot(p3_s[pl.ds(2 * I3, M3), :], w3_ref[2],
                        preferred_element_type=jnp.float32))
        act3_s[...] = jnp.maximum(a3 + b3_ref[...], 0.0)
        xf = _pool2x2(act3_s, W3, I3, C3)

        ils = []
        for c in range(NC):
            prod = wi_ref[c] * xf
            t = jnp.sum(prod.reshape(W4, G * H4, C3), axis=0)
            u = jnp.sum(t.reshape(G, H4, C3), axis=1)
            ils.append(jnp.sum(u, axis=1, keepdims=True))
        il = jnp.concatenate(ils, axis=1)
        mo = jnp.maximum(
            jnp.dot(meta_ref[:, 0, :], wm_ref[...],
                    preferred_element_type=jnp.float32) + bm_ref[...], 0.0)
        ml = jnp.dot(mo, wmo_ref[...], preferred_element_type=jnp.float32)
        o_ref[:, 0, :] = ml + bo_ref[...] + il

    casc_shapes = []
    r_, l_ = G * H2, W2 * C1
    for radix in _cascade_steps(W2):
        r_, l_ = r_ * radix, l_ // radix
        casc_shapes.append(pltpu.VMEM((r_, l_), jnp.bfloat16))

    const2 = lambda b: (0, 0)
    const3 = lambda b: (0, 0, 0)
    out = pl.pallas_call(
        body,
        out_shape=jax.ShapeDtypeStruct((B, 1, NC), jnp.float32),
        grid=(B // G,),
        in_specs=[
            pl.BlockSpec((G, Cin, H, W), lambda b: (b, 0, 0, 0)),
            pl.BlockSpec((G, 1, NM), lambda b: (b, 0, 0)),
            pl.BlockSpec(t1.shape, const2),
            pl.BlockSpec(b1_full.shape, const2),
            pl.BlockSpec(w2r.shape, const3),
            pl.BlockSpec(b2.shape, const2),
            pl.BlockSpec(w3r.shape, const3),
            pl.BlockSpec(b3.shape, const2),
            pl.BlockSpec(wi_exp.shape, const3),
            pl.BlockSpec(w_meta.shape, const2),
            pl.BlockSpec(b_meta.shape, const2),
            pl.BlockSpec(w_meta_out.shape, const2),
            pl.BlockSpec(b_out.shape, const2),
        ],
        out_specs=pl.BlockSpec((G, 1, NC), lambda b: (b, 0, 0)),
        scratch_shapes=[
            pltpu.VMEM((M2 + 2 * I2, 3 * C1), jnp.bfloat16),
            pltpu.VMEM((M2, C2), jnp.float32),
            pltpu.VMEM((M3 + 2 * I3, 3 * C2), jnp.bfloat16),
            pltpu.VMEM((M3, C3), jnp.float32),
        ] + casc_shapes,
        compiler_params=pltpu.CompilerParams(
            dimension_semantics=("parallel",),
            vmem_limit_bytes=56 * 1024 * 1024),
    )(img_nchw, meta3, t1, b1_full, w2r, b2, w3r, b3, wi_exp,
      w_meta, b_meta, w_meta_out, b_out)
    return out.reshape(B, NC)
